# baseline (speedup 1.0000x reference)
import jax
import jax.numpy as jnp
from jax import lax
from jax.experimental import pallas as pl
from jax.experimental.pallas import tpu as pltpu
from jax.experimental.pallas import tpu_sc as plsc

MAX_NORM = 2.0

B = 16384
L = 50
D = 32
NC = 2
NS = 16
NW = NC * NS
BPW = B // NW
ROWS_PER_CHUNK = 2
CHUNK_IDX = ROWS_PER_CHUNK * L
NCHUNK = BPW // ROWS_PER_CHUNK
NBUF = 4
UROWS = BPW // 128


def _renorm_tags_body(x_ref, o_ref):
    x = x_ref[...]
    ssq = jnp.sum(x * x, axis=1, keepdims=True)
    scale = jnp.minimum(1.0, MAX_NORM / jnp.maximum(jnp.sqrt(ssq), 1e-7))
    o_ref[...] = x * scale


def _renorm_tags(tags_table):
    n = tags_table.shape[0]
    blk = 1000
    return pl.pallas_call(
        _renorm_tags_body,
        grid=(n // blk,),
        in_specs=[pl.BlockSpec((blk, D), lambda i: (i, 0))],
        out_specs=pl.BlockSpec((blk, D), lambda i: (i, 0)),
        out_shape=jax.ShapeDtypeStruct((n, D), jnp.float32),
    )(tags_table)


def _combine_body(s_ref, q_ref, u8_ref, oh_ref, o_ref):
    S = s_ref[...]
    Q = q_ref[...]
    U8 = u8_ref[...]
    oh = oh_ref[...]
    U = jnp.sum(U8 * oh[:, :, None], axis=1)
    qU = jnp.sum(U * U, axis=1, keepdims=True)
    scale = jnp.minimum(1.0, MAX_NORM / jnp.maximum(jnp.sqrt(qU), 1e-7))
    v = U * scale + S
    tot = jnp.sum(v * v, axis=1, keepdims=True)
    ssq = qU * scale * scale + jnp.sum(Q, axis=1, keepdims=True)
    o_ref[...] = jax.nn.sigmoid(0.5 * (tot - ssq))


def _combine(S, Q, U8, oh):
    blk = 256
    return pl.pallas_call(
        _combine_body,
        grid=(B // blk,),
        in_specs=[
            pl.BlockSpec((blk, D), lambda i: (i, 0)),
            pl.BlockSpec((blk, 16), lambda i: (i, 0)),
            pl.BlockSpec((blk, 8, D), lambda i: (i, 0, 0)),
            pl.BlockSpec((blk, 8), lambda i: (i, 0)),
        ],
        out_specs=pl.BlockSpec((blk, 1), lambda i: (i, 0)),
        out_shape=jax.ShapeDtypeStruct((B, 1), jnp.float32),
    )(S, Q, U8, oh)


def _sc_bag_body(tags_hbm, tidx_hbm, s_hbm, q_hbm,
                 idx_v, ring_v, s_stage, q_stage,
                 sem0, sem1, sem2, sem3):
    sems = (sem0, sem1, sem2, sem3)
    wid = lax.axis_index("s") * NC + lax.axis_index("c")
    base = wid * BPW
    cbase = wid * NCHUNK

    pltpu.sync_copy(tidx_hbm.at[pl.ds(cbase, NCHUNK)], idx_v)

    for b in range(NBUF):
        pltpu.async_copy(tags_hbm.at[idx_v.at[b]], ring_v.at[b], sems[b])

    zero = jnp.zeros((16,), jnp.float32)

    def outer(g, _):
        for b in range(NBUF):
            cidx = g * NBUF + b
            pltpu.make_async_copy(tags_hbm.at[idx_v.at[cidx]],
                                  ring_v.at[b], sems[b]).wait()
            for seg in range(ROWS_PER_CHUNK):
                @plsc.parallel_loop(seg * L, (seg + 1) * L, step=1,
                                    unroll=5, carry=(zero, zero, zero))
                def acc(r, c, _b=b):
                    a0, a1, q = c
                    v0 = ring_v[_b, r, 0:16]
                    v1 = ring_v[_b, r, 16:32]
                    return (a0 + v0, a1 + v1, q + v0 * v0 + v1 * v1)
                a0, a1, q = acc
                row = cidx * ROWS_PER_CHUNK + seg
                s_stage[row, 0:16] = a0
                s_stage[row, 16:32] = a1
                q_stage[row, 0:16] = q
            nxt = cidx + NBUF

            @pl.when(nxt < NCHUNK)
            def _issue(_b=b, _nxt=nxt):
                pltpu.async_copy(tags_hbm.at[idx_v.at[_nxt]],
                                 ring_v.at[_b], sems[_b])
        return 0

    lax.fori_loop(0, NCHUNK // NBUF, outer, 0)

    pltpu.sync_copy(s_stage, s_hbm.at[pl.ds(base, BPW)])
    pltpu.sync_copy(q_stage, q_hbm.at[pl.ds(base, BPW)])


def _sc_bag(tags_renormed, tidx2):
    mesh = plsc.VectorSubcoreMesh(core_axis_name="c", subcore_axis_name="s")
    fn = pl.kernel(
        _sc_bag_body,
        out_type=(
            jax.ShapeDtypeStruct((B, D), jnp.float32),
            jax.ShapeDtypeStruct((B, 16), jnp.float32),
        ),
        mesh=mesh,
        compiler_params=pltpu.CompilerParams(use_tc_tiling_on_sc=False),
        scratch_types=[
            pltpu.VMEM((NCHUNK, CHUNK_IDX), jnp.int32),
            pltpu.VMEM((NBUF, CHUNK_IDX, D), jnp.float32),
            pltpu.VMEM((BPW, D), jnp.float32),
            pltpu.VMEM((BPW, 16), jnp.float32),
            pltpu.SemaphoreType.DMA,
            pltpu.SemaphoreType.DMA,
            pltpu.SemaphoreType.DMA,
            pltpu.SemaphoreType.DMA,
        ],
    )
    return fn(tags_renormed, tidx2)


URB = 8


def _sc_user_body(utab_hbm, uid_hbm, u8_hbm, uid_v, usem):
    wid = lax.axis_index("s") * NC + lax.axis_index("c")
    base = wid * BPW
    lane = lax.iota(jnp.int32, 16)

    pltpu.sync_copy(uid_hbm.at[pl.ds(base, BPW)], uid_v)

    def uid_at(i):
        grp = (i // 16) * 16
        u16 = uid_v[pl.ds(grp, 16)]
        return jnp.sum(jnp.where(lane == i - grp, u16, 0))

    def drain():
        pltpu.make_async_copy(utab_hbm.at[0], u8_hbm.at[0], usem).wait()

    def outer(i, _):
        u = uid_at(i)
        tile = lax.shift_right_logical(u, 3)
        pltpu.async_copy(utab_hbm.at[tile], u8_hbm.at[base + i], usem)

        @pl.when(i >= URB)
        def _d():
            drain()
        return 0

    lax.fori_loop(0, BPW, outer, 0)
    for _ in range(URB):
        drain()


def _sc_user_gather(users3, user_id):
    mesh = plsc.VectorSubcoreMesh(core_axis_name="c", subcore_axis_name="s")
    fn = pl.kernel(
        _sc_user_body,
        out_type=jax.ShapeDtypeStruct((B, 8, D), jnp.float32),
        mesh=mesh,
        compiler_params=pltpu.CompilerParams(use_tc_tiling_on_sc=True,
                                             needs_layout_passes=False),
        scratch_types=[
            pltpu.VMEM((BPW,), jnp.int32),
            pltpu.SemaphoreType.DMA,
        ],
    )
    return fn(users3, user_id)


def kernel(user_id, tag_ids, users_table, tags_table):
    n_users = users_table.shape[0]
    users3 = users_table.reshape(n_users // 8, 8, D)
    tags_rn = _renorm_tags(tags_table)
    tidx2 = tag_ids.reshape(B // ROWS_PER_CHUNK, CHUNK_IDX).astype(jnp.int32)
    S, Q = _sc_bag(tags_rn, tidx2)
    uid32 = user_id.astype(jnp.int32)
    U8 = _sc_user_gather(users3, uid32)
    oh = jax.nn.one_hot(jnp.bitwise_and(uid32, 7), 8, dtype=jnp.float32)
    return _combine(S, Q, U8, oh).reshape(B)


# speedup vs baseline: 3.4203x; 3.4203x over previous
import jax
import jax.numpy as jnp
from jax import lax
from jax.experimental import pallas as pl
from jax.experimental.pallas import tpu as pltpu
from jax.experimental.pallas import tpu_sc as plsc

MAX_NORM = 2.0

B = 16384
L = 50
D = 32
NC = 2
NS = 16
NW = NC * NS
BPW = B // NW
ROWS_PER_CHUNK = 2
CHUNK_IDX = ROWS_PER_CHUNK * L
NCHUNK = BPW // ROWS_PER_CHUNK
NBUF = 4
UROWS = BPW // 128


def _renorm_tags_body(x_ref, o_ref):
    x = x_ref[...]
    ssq = jnp.sum(x * x, axis=1, keepdims=True)
    scale = jnp.minimum(1.0, MAX_NORM / jnp.maximum(jnp.sqrt(ssq), 1e-7))
    o_ref[...] = x * scale


def _renorm_tags(tags_table):
    n = tags_table.shape[0]
    blk = 1000
    return pl.pallas_call(
        _renorm_tags_body,
        grid=(n // blk,),
        in_specs=[pl.BlockSpec((blk, D), lambda i: (i, 0))],
        out_specs=pl.BlockSpec((blk, D), lambda i: (i, 0)),
        out_shape=jax.ShapeDtypeStruct((n, D), jnp.float32),
    )(tags_table)


def _combine_body(s_ref, q_ref, u8_ref, oh_ref, o_ref):
    S = s_ref[...]
    Q = q_ref[...]
    U = jnp.sum(u8_ref[...] * oh_ref[...][:, :, None], axis=1)
    qU = jnp.sum(U * U, axis=1, keepdims=True)
    scale = jnp.minimum(1.0, MAX_NORM / jnp.maximum(jnp.sqrt(qU), 1e-7))
    v = U * scale + S
    tot = jnp.sum(v * v, axis=1, keepdims=True)
    ssq = qU * scale * scale + jnp.sum(Q, axis=1, keepdims=True)
    o_ref[...] = jax.nn.sigmoid(0.5 * (tot - ssq))


def _combine(S, Q, U8, oh):
    blk = 256
    return pl.pallas_call(
        _combine_body,
        grid=(B // blk,),
        in_specs=[
            pl.BlockSpec((blk, D), lambda i: (i, 0)),
            pl.BlockSpec((blk, 16), lambda i: (i, 0)),
            pl.BlockSpec((blk, 8, D), lambda i: (i, 0, 0)),
            pl.BlockSpec((blk, 8), lambda i: (i, 0)),
        ],
        out_specs=pl.BlockSpec((blk, 1), lambda i: (i, 0)),
        out_shape=jax.ShapeDtypeStruct((B, 1), jnp.float32),
    )(S, Q, U8, oh)


def _sc_bag_body(tags_hbm, tidx_hbm, s_hbm, q_hbm,
                 idx_v, ring_v, s_stage, q_stage,
                 sem0, sem1, sem2, sem3):
    sems = (sem0, sem1, sem2, sem3)
    wid = lax.axis_index("s") * NC + lax.axis_index("c")
    base = wid * BPW
    cbase = wid * NCHUNK

    pltpu.sync_copy(tidx_hbm.at[pl.ds(cbase, NCHUNK)], idx_v)

    for b in range(NBUF):
        pltpu.async_copy(tags_hbm.at[idx_v.at[b]], ring_v.at[b], sems[b])

    zero = jnp.zeros((16,), jnp.float32)

    def outer(g, _):
        for b in range(NBUF):
            cidx = g * NBUF + b
            pltpu.make_async_copy(tags_hbm.at[idx_v.at[cidx]],
                                  ring_v.at[b], sems[b]).wait()
            for seg in range(ROWS_PER_CHUNK):
                @plsc.parallel_loop(seg * L, (seg + 1) * L, step=1,
                                    unroll=5, carry=(zero, zero, zero))
                def acc(r, c, _b=b):
                    a0, a1, q = c
                    v0 = ring_v[_b, r, 0:16]
                    v1 = ring_v[_b, r, 16:32]
                    return (a0 + v0, a1 + v1, q + v0 * v0 + v1 * v1)
                a0, a1, q = acc
                row = cidx * ROWS_PER_CHUNK + seg
                s_stage[row, 0:16] = a0
                s_stage[row, 16:32] = a1
                q_stage[row, 0:16] = q
            nxt = cidx + NBUF

            @pl.when(nxt < NCHUNK)
            def _issue(_b=b, _nxt=nxt):
                pltpu.async_copy(tags_hbm.at[idx_v.at[_nxt]],
                                 ring_v.at[_b], sems[_b])
        return 0

    lax.fori_loop(0, NCHUNK // NBUF, outer, 0)

    pltpu.sync_copy(s_stage, s_hbm.at[pl.ds(base, BPW)])
    pltpu.sync_copy(q_stage, q_hbm.at[pl.ds(base, BPW)])


def _sc_bag(tags_renormed, tidx2):
    mesh = plsc.VectorSubcoreMesh(core_axis_name="c", subcore_axis_name="s")
    fn = pl.kernel(
        _sc_bag_body,
        out_type=(
            jax.ShapeDtypeStruct((B, D), jnp.float32),
            jax.ShapeDtypeStruct((B, 16), jnp.float32),
        ),
        mesh=mesh,
        compiler_params=pltpu.CompilerParams(use_tc_tiling_on_sc=False),
        scratch_types=[
            pltpu.VMEM((NCHUNK, CHUNK_IDX), jnp.int32),
            pltpu.VMEM((NBUF, CHUNK_IDX, D), jnp.float32),
            pltpu.VMEM((BPW, D), jnp.float32),
            pltpu.VMEM((BPW, 16), jnp.float32),
            pltpu.SemaphoreType.DMA,
            pltpu.SemaphoreType.DMA,
            pltpu.SemaphoreType.DMA,
            pltpu.SemaphoreType.DMA,
        ],
    )
    return fn(tags_renormed, tidx2)


URB = 8


UGC = 128
NUG = BPW // UGC


def _sc_user_body(utab_hbm, uid_hbm, u8_hbm,
                  uid_v, idx_v, gbuf0, gbuf1, us0, us1):
    gbufs = (gbuf0, gbuf1)
    usems = (us0, us1)
    wid = lax.axis_index("s") * NC + lax.axis_index("c")
    base = wid * BPW

    pltpu.sync_copy(uid_hbm.at[pl.ds(base, BPW)], uid_v)
    for r in range(NUG):
        for h in range(UGC // 16):
            u16 = uid_v[pl.ds(r * UGC + h * 16, 16)]
            idx_v[r, pl.ds(h * 16, 16)] = lax.shift_right_logical(u16, 3)

    for r in range(2):
        pltpu.async_copy(utab_hbm.at[idx_v.at[r]], gbufs[r], usems[r])

    for r in range(NUG):
        b = r % 2
        pltpu.make_async_copy(utab_hbm.at[idx_v.at[r]],
                              gbufs[b], usems[b]).wait()
        pltpu.sync_copy(gbufs[b], u8_hbm.at[pl.ds(base + r * UGC, UGC)])
        nr = r + 2
        if nr < NUG:
            pltpu.async_copy(utab_hbm.at[idx_v.at[nr]], gbufs[b], usems[b])


def _sc_user_gather(users3, user_id):
    mesh = plsc.VectorSubcoreMesh(core_axis_name="c", subcore_axis_name="s")
    fn = pl.kernel(
        _sc_user_body,
        out_type=jax.ShapeDtypeStruct((B, 8, D), jnp.float32),
        mesh=mesh,
        compiler_params=pltpu.CompilerParams(use_tc_tiling_on_sc=False),
        scratch_types=[
            pltpu.VMEM((BPW,), jnp.int32),
            pltpu.VMEM((NUG, UGC), jnp.int32),
            pltpu.VMEM((UGC, 8, D), jnp.float32),
            pltpu.VMEM((UGC, 8, D), jnp.float32),
            pltpu.SemaphoreType.DMA,
            pltpu.SemaphoreType.DMA,
        ],
    )
    return fn(users3, user_id)


def kernel(user_id, tag_ids, users_table, tags_table):
    n_users = users_table.shape[0]
    users3 = users_table.reshape(n_users // 8, 8, D)
    tags_rn = _renorm_tags(tags_table)
    tidx2 = tag_ids.reshape(B // ROWS_PER_CHUNK, CHUNK_IDX).astype(jnp.int32)
    S, Q = _sc_bag(tags_rn, tidx2)
    uid32 = user_id.astype(jnp.int32)
    U8 = _sc_user_gather(users3, uid32)
    oh = jax.nn.one_hot(jnp.bitwise_and(uid32, 7), 8, dtype=jnp.float32)
    return _combine(S, Q, U8, oh).reshape(B)


# speedup vs baseline: 3.9835x; 1.1647x over previous
import jax
import jax.numpy as jnp
from jax import lax
from jax.experimental import pallas as pl
from jax.experimental.pallas import tpu as pltpu
from jax.experimental.pallas import tpu_sc as plsc

MAX_NORM = 2.0

B = 16384
L = 50
D = 32
NC = 2
NS = 16
NW = NC * NS
BPW = B // NW
ROWS_PER_CHUNK = 2
CHUNK_IDX = ROWS_PER_CHUNK * L
NCHUNK = BPW // ROWS_PER_CHUNK
NBUF = 8
UROWS = BPW // 128


def _renorm_tags_body(x_ref, o_ref):
    x = x_ref[...]
    r = lax.broadcasted_iota(jnp.int32, (128, 128), 0) // D
    c = lax.broadcasted_iota(jnp.int32, (128, 128), 1) // D
    m = (r == c).astype(jnp.float32)
    ssq = jax.lax.dot(x * x, m, precision=jax.lax.Precision.HIGHEST)
    scale = jnp.minimum(1.0, MAX_NORM / jnp.maximum(jnp.sqrt(ssq), 1e-7))
    o_ref[...] = x * scale


def _renorm_tags(tags_packed):
    n4 = tags_packed.shape[0]
    blk = 1000
    return pl.pallas_call(
        _renorm_tags_body,
        grid=(n4 // blk,),
        in_specs=[pl.BlockSpec((blk, 128), lambda i: (i, 0))],
        out_specs=pl.BlockSpec((blk, 128), lambda i: (i, 0)),
        out_shape=jax.ShapeDtypeStruct((n4, 128), jnp.float32),
    )(tags_packed)


def _combine_body(s_ref, q_ref, u_ref, o_ref):
    S = s_ref[...]
    Q = q_ref[...]
    U = u_ref[...]
    qU = jnp.sum(U * U, axis=1, keepdims=True)
    scale = jnp.minimum(1.0, MAX_NORM / jnp.maximum(jnp.sqrt(qU), 1e-7))
    v = U * scale + S
    tot = jnp.sum(v * v, axis=1, keepdims=True)
    ssq = qU * scale * scale + jnp.sum(Q, axis=1, keepdims=True)
    o_ref[...] = jax.nn.sigmoid(0.5 * (tot - ssq))


def _combine(S, Q, U):
    blk = 256
    return pl.pallas_call(
        _combine_body,
        grid=(B // blk,),
        in_specs=[
            pl.BlockSpec((blk, D), lambda i: (i, 0)),
            pl.BlockSpec((blk, 16), lambda i: (i, 0)),
            pl.BlockSpec((blk, D), lambda i: (i, 0)),
        ],
        out_specs=pl.BlockSpec((blk, 1), lambda i: (i, 0)),
        out_shape=jax.ShapeDtypeStruct((B, 1), jnp.float32),
    )(S, Q, U)


def _sc_bag_body(tags_hbm, tidx_hbm, s_hbm, q_hbm,
                 idx_v, ring_v, s_stage, q_stage, *sems):
    wid = lax.axis_index("s") * NC + lax.axis_index("c")
    base = wid * BPW
    cbase = wid * NCHUNK

    pltpu.sync_copy(tidx_hbm.at[pl.ds(cbase, NCHUNK)], idx_v)

    for b in range(NBUF):
        pltpu.async_copy(tags_hbm.at[idx_v.at[b]], ring_v.at[b], sems[b])

    zero = jnp.zeros((16,), jnp.float32)

    def outer(g, _):
        for b in range(NBUF):
            cidx = g * NBUF + b
            pltpu.make_async_copy(tags_hbm.at[idx_v.at[cidx]],
                                  ring_v.at[b], sems[b]).wait()
            for seg in range(ROWS_PER_CHUNK):
                @plsc.parallel_loop(seg * L, (seg + 1) * L, step=1,
                                    unroll=10, carry=(zero, zero, zero))
                def acc(r, c, _b=b):
                    a0, a1, q = c
                    v0 = ring_v[_b, r, 0:16]
                    v1 = ring_v[_b, r, 16:32]
                    return (a0 + v0, a1 + v1, q + v0 * v0 + v1 * v1)
                a0, a1, q = acc
                row = cidx * ROWS_PER_CHUNK + seg
                s_stage[row, 0:16] = a0
                s_stage[row, 16:32] = a1
                q_stage[row, 0:16] = q
            nxt = cidx + NBUF

            @pl.when(nxt < NCHUNK)
            def _issue(_b=b, _nxt=nxt):
                pltpu.async_copy(tags_hbm.at[idx_v.at[_nxt]],
                                 ring_v.at[_b], sems[_b])
        return 0

    lax.fori_loop(0, NCHUNK // NBUF, outer, 0)

    pltpu.sync_copy(s_stage, s_hbm.at[pl.ds(base, BPW)])
    pltpu.sync_copy(q_stage, q_hbm.at[pl.ds(base, BPW)])


def _sc_bag(tags_renormed, tidx2):
    mesh = plsc.VectorSubcoreMesh(core_axis_name="c", subcore_axis_name="s")
    fn = pl.kernel(
        _sc_bag_body,
        out_type=(
            jax.ShapeDtypeStruct((B, D), jnp.float32),
            jax.ShapeDtypeStruct((B, 16), jnp.float32),
        ),
        mesh=mesh,
        compiler_params=pltpu.CompilerParams(use_tc_tiling_on_sc=False),
        scratch_types=[
            pltpu.VMEM((NCHUNK, CHUNK_IDX), jnp.int32),
            pltpu.VMEM((NBUF, CHUNK_IDX, D), jnp.float32),
            pltpu.VMEM((BPW, D), jnp.float32),
            pltpu.VMEM((BPW, 16), jnp.float32),
        ] + [pltpu.SemaphoreType.DMA] * NBUF,
    )
    return fn(tags_renormed, tidx2)


URB = 8


def _sc_user_body(utab_hbm, uid_hbm, u_hbm, uidx_v, u_stage, usem):
    wid = lax.axis_index("s") * NC + lax.axis_index("c")
    base = wid * BPW
    pltpu.sync_copy(uid_hbm.at[pl.ds(wid * UROWS, UROWS)], uidx_v)
    for j in range(UROWS):
        pltpu.async_copy(utab_hbm.at[uidx_v.at[j]],
                         u_stage.at[pl.ds(j * 128, 128)], usem)
    for j in range(UROWS):
        pltpu.make_async_copy(utab_hbm.at[uidx_v.at[j]],
                              u_stage.at[pl.ds(j * 128, 128)], usem).wait()
    pltpu.sync_copy(u_stage, u_hbm.at[pl.ds(base, BPW)])


def _sc_user_gather(users2, uid2):
    mesh = plsc.VectorSubcoreMesh(core_axis_name="c", subcore_axis_name="s")
    fn = pl.kernel(
        _sc_user_body,
        out_type=jax.ShapeDtypeStruct((B, D), jnp.float32),
        mesh=mesh,
        compiler_params=pltpu.CompilerParams(use_tc_tiling_on_sc=False),
        scratch_types=[
            pltpu.VMEM((UROWS, 128), jnp.int32),
            pltpu.VMEM((BPW, D), jnp.float32),
            pltpu.SemaphoreType.DMA,
        ],
    )
    return fn(users2, uid2)


def kernel(user_id, tag_ids, users_table, tags_table):
    n_users = users_table.shape[0]
    users_flat = lax.optimization_barrier(users_table.reshape(-1))
    users2 = users_flat.reshape(n_users, D)
    n_tags = tags_table.shape[0]
    tags_rn = _renorm_tags(tags_table.reshape(n_tags // 4, 4 * D))
    tags_rn = tags_rn.reshape(n_tags, D)
    tidx2 = tag_ids.reshape(B // ROWS_PER_CHUNK, CHUNK_IDX).astype(jnp.int32)
    uid2 = user_id.reshape(B // 128, 128).astype(jnp.int32)
    S, Q = _sc_bag(tags_rn, tidx2)
    U = _sc_user_gather(users2, uid2)
    return _combine(S, Q, U).reshape(B)


# speedup vs baseline: 4.1507x; 1.0420x over previous
import jax
import jax.numpy as jnp
from jax import lax
from jax.experimental import pallas as pl
from jax.experimental.pallas import tpu as pltpu
from jax.experimental.pallas import tpu_sc as plsc

MAX_NORM = 2.0

B = 16384
L = 50
D = 32
NC = 2
NS = 16
NW = NC * NS
BPW = B // NW
ROWS_PER_CHUNK = 2
CHUNK_IDX = ROWS_PER_CHUNK * L
NCHUNK = BPW // ROWS_PER_CHUNK
NBUF = 8
UROWS = BPW // 128


def _renorm_tags_body(x_ref, o_ref):
    x = x_ref[...]
    r = lax.broadcasted_iota(jnp.int32, (128, 128), 0) // D
    c = lax.broadcasted_iota(jnp.int32, (128, 128), 1) // D
    m = (r == c).astype(jnp.float32)
    ssq = jax.lax.dot(x * x, m, precision=jax.lax.Precision.HIGHEST)
    scale = jnp.minimum(1.0, MAX_NORM / jnp.maximum(jnp.sqrt(ssq), 1e-7))
    o_ref[...] = x * scale


def _renorm_tags(tags_packed):
    n4 = tags_packed.shape[0]
    blk = 1000
    return pl.pallas_call(
        _renorm_tags_body,
        grid=(n4 // blk,),
        in_specs=[pl.BlockSpec((blk, 128), lambda i: (i, 0))],
        out_specs=pl.BlockSpec((blk, 128), lambda i: (i, 0)),
        out_shape=jax.ShapeDtypeStruct((n4, 128), jnp.float32),
    )(tags_packed)


def _combine_body(s_ref, q_ref, u_ref, o_ref):
    S = s_ref[...]
    Q = q_ref[...]
    U = u_ref[...]
    qU = jnp.sum(U * U, axis=1, keepdims=True)
    scale = jnp.minimum(1.0, MAX_NORM / jnp.maximum(jnp.sqrt(qU), 1e-7))
    v = U * scale + S
    tot = jnp.sum(v * v, axis=1, keepdims=True)
    ssq = qU * scale * scale + jnp.sum(Q, axis=1, keepdims=True)
    o_ref[...] = jax.nn.sigmoid(0.5 * (tot - ssq))


def _combine(S, Q, U):
    blk = 1024
    return pl.pallas_call(
        _combine_body,
        grid=(B // blk,),
        in_specs=[
            pl.BlockSpec((blk, D), lambda i: (i, 0)),
            pl.BlockSpec((blk, 16), lambda i: (i, 0)),
            pl.BlockSpec((blk, D), lambda i: (i, 0)),
        ],
        out_specs=pl.BlockSpec((blk, 1), lambda i: (i, 0)),
        out_shape=jax.ShapeDtypeStruct((B, 1), jnp.float32),
    )(S, Q, U)


def _sc_bag_body(tags_hbm, tidx_hbm, s_hbm, q_hbm,
                 idx_v, ring_v, s_stage, q_stage, *sems):
    wid = lax.axis_index("s") * NC + lax.axis_index("c")
    base = wid * BPW
    cbase = wid * NCHUNK

    pltpu.sync_copy(tidx_hbm.at[pl.ds(cbase, NCHUNK)], idx_v)

    for b in range(NBUF):
        pltpu.async_copy(tags_hbm.at[idx_v.at[b]], ring_v.at[b], sems[b])

    zero = jnp.zeros((16,), jnp.float32)

    def outer(g, _):
        for b in range(NBUF):
            cidx = g * NBUF + b
            pltpu.make_async_copy(tags_hbm.at[idx_v.at[cidx]],
                                  ring_v.at[b], sems[b]).wait()
            for seg in range(ROWS_PER_CHUNK):
                @plsc.parallel_loop(seg * L, (seg + 1) * L, step=1,
                                    unroll=10, carry=(zero, zero, zero))
                def acc(r, c, _b=b):
                    a0, a1, q = c
                    v0 = ring_v[_b, r, 0:16]
                    v1 = ring_v[_b, r, 16:32]
                    return (a0 + v0, a1 + v1, q + v0 * v0 + v1 * v1)
                a0, a1, q = acc
                row = cidx * ROWS_PER_CHUNK + seg
                s_stage[row, 0:16] = a0
                s_stage[row, 16:32] = a1
                q_stage[row, 0:16] = q
            nxt = cidx + NBUF

            @pl.when(nxt < NCHUNK)
            def _issue(_b=b, _nxt=nxt):
                pltpu.async_copy(tags_hbm.at[idx_v.at[_nxt]],
                                 ring_v.at[_b], sems[_b])
        return 0

    lax.fori_loop(0, NCHUNK // NBUF, outer, 0)

    pltpu.sync_copy(s_stage, s_hbm.at[pl.ds(base, BPW)])
    pltpu.sync_copy(q_stage, q_hbm.at[pl.ds(base, BPW)])


def _sc_bag(tags_renormed, tidx2):
    mesh = plsc.VectorSubcoreMesh(core_axis_name="c", subcore_axis_name="s")
    fn = pl.kernel(
        _sc_bag_body,
        out_type=(
            jax.ShapeDtypeStruct((B, D), jnp.float32),
            jax.ShapeDtypeStruct((B, 16), jnp.float32),
        ),
        mesh=mesh,
        compiler_params=pltpu.CompilerParams(use_tc_tiling_on_sc=False),
        scratch_types=[
            pltpu.VMEM((NCHUNK, CHUNK_IDX), jnp.int32),
            pltpu.VMEM((NBUF, CHUNK_IDX, D), jnp.float32),
            pltpu.VMEM((BPW, D), jnp.float32),
            pltpu.VMEM((BPW, 16), jnp.float32),
        ] + [pltpu.SemaphoreType.DMA] * NBUF,
    )
    return fn(tags_renormed, tidx2)


URB = 8


def _sc_user_body(utab_hbm, uid_hbm, u_hbm, uidx_v, u_stage, usem):
    wid = lax.axis_index("s") * NC + lax.axis_index("c")
    base = wid * BPW
    pltpu.sync_copy(uid_hbm.at[pl.ds(wid * UROWS, UROWS)], uidx_v)
    for j in range(UROWS):
        pltpu.async_copy(utab_hbm.at[uidx_v.at[j]],
                         u_stage.at[pl.ds(j * 128, 128)], usem)
    for j in range(UROWS):
        pltpu.make_async_copy(utab_hbm.at[uidx_v.at[j]],
                              u_stage.at[pl.ds(j * 128, 128)], usem).wait()
    pltpu.sync_copy(u_stage, u_hbm.at[pl.ds(base, BPW)])


def _sc_user_gather(users2, uid2):
    mesh = plsc.VectorSubcoreMesh(core_axis_name="c", subcore_axis_name="s")
    fn = pl.kernel(
        _sc_user_body,
        out_type=jax.ShapeDtypeStruct((B, D), jnp.float32),
        mesh=mesh,
        compiler_params=pltpu.CompilerParams(use_tc_tiling_on_sc=False),
        scratch_types=[
            pltpu.VMEM((UROWS, 128), jnp.int32),
            pltpu.VMEM((BPW, D), jnp.float32),
            pltpu.SemaphoreType.DMA,
        ],
    )
    return fn(users2, uid2)


def kernel(user_id, tag_ids, users_table, tags_table):
    n_users = users_table.shape[0]
    users2 = users_table.reshape(n_users, D)
    n_tags = tags_table.shape[0]
    tags_rn = _renorm_tags(tags_table.reshape(n_tags // 4, 4 * D))
    tags_rn = tags_rn.reshape(n_tags, D)
    tidx2 = tag_ids.reshape(B // ROWS_PER_CHUNK, CHUNK_IDX).astype(jnp.int32)
    uid2 = user_id.reshape(B // 128, 128).astype(jnp.int32)
    S, Q = _sc_bag(tags_rn, tidx2)
    U = _sc_user_gather(users2, uid2)
    return _combine(S, Q, U).reshape(B)
